# Initial kernel scaffold; baseline (speedup 1.0000x reference)
import jax
import jax.numpy as jnp
from jax.experimental import pallas as pl


def kernel(input_, table):
    raise NotImplementedError("write your pallas kernel here")


# speedup vs baseline: 3.5745x; 3.5745x over previous
import functools

import jax
import jax.numpy as jnp
from jax import lax
from jax.experimental import pallas as pl
from jax.experimental.pallas import tpu as pltpu
from jax.experimental.pallas import tpu_sc as plsc

N_V = 1000
N_D = 64
B = 4096
T = 200
TOT = B * T
NW = 32
PER_W = TOT // NW
C = 128
NCH = PER_W // C
NBUF = 4
ROUNDS = NCH // NBUF


@jax.jit
def _sc_embedding_lookup(idx3, table):
  mesh = plsc.VectorSubcoreMesh(core_axis_name="c", subcore_axis_name="s")

  @functools.partial(
      pl.kernel,
      mesh=mesh,
      out_type=jax.ShapeDtypeStruct((TOT, N_D), jnp.float32),
      compiler_params=pltpu.CompilerParams(use_tc_tiling_on_sc=False),
      scratch_types=(
          [pltpu.VMEM((NCH, C), jnp.int32)]
          + [pltpu.VMEM((C, N_D), jnp.float32) for _ in range(NBUF)]
          + [pltpu.SemaphoreType.DMA for _ in range(2 * NBUF)]
      ),
  )
  def k(idx_hbm, table_hbm, out_hbm, idx_v, *bufs_and_sems):
    rows = bufs_and_sems[:NBUF]
    gsems = bufs_and_sems[NBUF:2 * NBUF]
    ssems = bufs_and_sems[2 * NBUF:]

    wid = lax.axis_index("s") * 2 + lax.axis_index("c")
    row0 = wid * PER_W

    pltpu.sync_copy(idx_hbm.at[wid], idx_v)

    for b in range(NBUF):
      pltpu.async_copy(table_hbm.at[idx_v.at[b]], rows[b], gsems[b])

    def round_body(r, carry):
      j0 = r * NBUF
      for b in range(NBUF):
        j = j0 + b
        pltpu.make_async_copy(
            table_hbm.at[idx_v.at[j]], rows[b], gsems[b]).wait()
        pltpu.async_copy(
            rows[b], out_hbm.at[pl.ds(row0 + j * C, C)], ssems[b])
      for b in range(NBUF):
        j = j0 + b
        pltpu.make_async_copy(
            rows[b], out_hbm.at[pl.ds(row0 + j * C, C)], ssems[b]).wait()
        nj = j + NBUF

        @pl.when(nj < NCH)
        def _():
          pltpu.async_copy(table_hbm.at[idx_v.at[nj]], rows[b], gsems[b])

      return carry

    lax.fori_loop(0, ROUNDS, round_body, 0)

  return k(idx3, table)


def kernel(input_, table):
  idx3 = input_.reshape(NW, NCH, C)
  out = _sc_embedding_lookup(idx3, table)
  return out.reshape(B, T, N_D)


# speedup vs baseline: 3.5858x; 1.0032x over previous
import functools

import jax
import jax.numpy as jnp
from jax import lax
from jax.experimental import pallas as pl
from jax.experimental.pallas import tpu as pltpu
from jax.experimental.pallas import tpu_sc as plsc

N_V = 1000
N_D = 64
B = 4096
T = 200
TOT = B * T
NW = 32
PER_W = TOT // NW
C = 256
NCH = PER_W // C
NBUF = 4
ROUNDS = NCH // NBUF


@jax.jit
def _sc_embedding_lookup(idx3, table):
  mesh = plsc.VectorSubcoreMesh(core_axis_name="c", subcore_axis_name="s")

  @functools.partial(
      pl.kernel,
      mesh=mesh,
      out_type=jax.ShapeDtypeStruct((TOT, N_D), jnp.float32),
      compiler_params=pltpu.CompilerParams(use_tc_tiling_on_sc=False),
      scratch_types=(
          [pltpu.VMEM((NCH, C), jnp.int32)]
          + [pltpu.VMEM((C, N_D), jnp.float32) for _ in range(NBUF)]
          + [pltpu.SemaphoreType.DMA for _ in range(2 * NBUF)]
      ),
  )
  def k(idx_hbm, table_hbm, out_hbm, idx_v, *bufs_and_sems):
    rows = bufs_and_sems[:NBUF]
    gsems = bufs_and_sems[NBUF:2 * NBUF]
    ssems = bufs_and_sems[2 * NBUF:]

    wid = lax.axis_index("s") * 2 + lax.axis_index("c")
    row0 = wid * PER_W

    pltpu.sync_copy(idx_hbm.at[wid], idx_v)

    for b in range(NBUF):
      pltpu.async_copy(table_hbm.at[idx_v.at[b]], rows[b], gsems[b])

    def round_body(r, carry):
      j0 = r * NBUF
      for b in range(NBUF):
        j = j0 + b
        pltpu.make_async_copy(
            table_hbm.at[idx_v.at[j]], rows[b], gsems[b]).wait()
        pltpu.async_copy(
            rows[b], out_hbm.at[pl.ds(row0 + j * C, C)], ssems[b])
      for b in range(NBUF):
        j = j0 + b
        pltpu.make_async_copy(
            rows[b], out_hbm.at[pl.ds(row0 + j * C, C)], ssems[b]).wait()
        nj = j + NBUF

        @pl.when(nj < NCH)
        def _():
          pltpu.async_copy(table_hbm.at[idx_v.at[nj]], rows[b], gsems[b])

      return carry

    lax.fori_loop(0, ROUNDS, round_body, 0)

  return k(idx3, table)


def kernel(input_, table):
  idx3 = input_.reshape(NW, NCH, C)
  out = _sc_embedding_lookup(idx3, table)
  return out.reshape(B, T, N_D)
